# Initial kernel scaffold; baseline (speedup 1.0000x reference)
import jax
import jax.numpy as jnp
from jax.experimental import pallas as pl


def kernel(x, edge_index, eps1, W1a, b1a, g1, be1, W1b, b1b, eps2, W2a, b2a, g2, be2, W2b, b2b):
    raise NotImplementedError("write your pallas kernel here")


# speedup vs baseline: 2.9063x; 2.9063x over previous
import functools

import jax
import jax.numpy as jnp
from jax import lax
from jax.experimental import pallas as pl
from jax.experimental.pallas import tpu as pltpu
from jax.experimental.pallas import tpu_sc as plsc

N = 10000
D = 128
E = 320000

NC = 2
NS = 16
NW = NC * NS
K = 128
CHUNKS = (E + NW * K - 1) // (NW * K)
E_PAD = NW * K * 80
CHUNKS = 80
ROWS_PT = 632
ACC_ROWS = NS * ROWS_PT


def _sc_agg_body(x_hbm, src_hbm, dst_hbm, out_hbm, src_v, dst_v, rows_v,
                 zbuf, acc, sem):
    c = lax.axis_index("c")
    s = lax.axis_index("s")
    w = c * NS + s

    zeros16 = jnp.zeros((16,), jnp.float32)
    for i in range(16):
        for jj in range(D // 16):
            zbuf[i, pl.ds(jj * 16, 16)] = zeros16
    zbase = s * ROWS_PT

    def zloop(k, carry):
        pltpu.sync_copy(zbuf, acc.at[pl.ds(zbase + k * 16, 16)])
        return carry

    lax.fori_loop(0, ROWS_PT // 16, zloop, 0)
    pltpu.sync_copy(zbuf.at[pl.ds(0, 8)],
                    acc.at[pl.ds(zbase + (ROWS_PT // 16) * 16, 8)])
    plsc.subcore_barrier()

    pltpu.sync_copy(src_hbm.at[w], src_v)
    pltpu.sync_copy(dst_hbm.at[w], dst_v)

    def body(j, carry):
        pltpu.async_copy(x_hbm.at[src_v.at[j]], rows_v, sem).wait()
        pltpu.sync_copy(rows_v, acc.at[dst_v.at[j]], add=True)
        return carry

    lax.fori_loop(0, CHUNKS, body, 0)
    plsc.subcore_barrier()

    pltpu.sync_copy(acc.at[pl.ds(s * ROWS_PT, ROWS_PT)],
                    out_hbm.at[c, pl.ds(s * ROWS_PT, ROWS_PT)])


@functools.cache
def _get_sc_agg():
    return functools.partial(
        pl.kernel,
        out_type=jax.ShapeDtypeStruct((NC, ACC_ROWS, D), jnp.float32),
        mesh=plsc.VectorSubcoreMesh(core_axis_name="c", subcore_axis_name="s",
                                    num_cores=NC, num_subcores=NS),
        scratch_types=[
            pltpu.VMEM((CHUNKS, K), jnp.int32),
            pltpu.VMEM((CHUNKS, K), jnp.int32),
            pltpu.VMEM((K, D), jnp.float32),
            pltpu.VMEM((16, D), jnp.float32),
            pltpu.VMEM_SHARED((ACC_ROWS, D), jnp.float32),
            pltpu.SemaphoreType.DMA,
        ],
    )(_sc_agg_body)


def _mlp_body(eps_ref, x_ref, p0_ref, p1_ref, wa_ref, ba_ref, g_ref, be_ref,
              wb_ref, bb_ref, o_ref, *, final_act):
    h0 = x_ref[...] * (1.0 + eps_ref[0]) + p0_ref[0] + p1_ref[0]
    h = lax.dot_general(h0, wa_ref[...], (((1,), (0,)), ((), ())),
                        precision=lax.Precision.HIGHEST,
                        preferred_element_type=jnp.float32) + ba_ref[...]
    mu = jnp.mean(h, axis=-1, keepdims=True)
    var = jnp.mean((h - mu) ** 2, axis=-1, keepdims=True)
    h = (h - mu) / jnp.sqrt(var + 1e-5) * g_ref[...] + be_ref[...]
    h = jnp.where(h > 0, h, 0.01 * h)
    h = lax.dot_general(h, wb_ref[...], (((1,), (0,)), ((), ())),
                        precision=lax.Precision.HIGHEST,
                        preferred_element_type=jnp.float32) + bb_ref[...]
    nrm = jnp.sqrt(jnp.sum(h * h, axis=-1, keepdims=True))
    h = h / jnp.maximum(nrm, 1e-12)
    if final_act:
        h = jnp.where(h > 0, h, 0.01 * h)
    o_ref[...] = h


def _tc_layer(eps, x, parts, waT, ba, g, be, wbT, bb, final_act, interpret=False):
    R = 2000
    grid = (N // R,)
    row_spec = pl.BlockSpec((R, D), lambda i: (i, 0))
    part0_spec = pl.BlockSpec((1, R, D), lambda i: (0, i, 0))
    part1_spec = pl.BlockSpec((1, R, D), lambda i: (1, i, 0))
    full_spec = pl.BlockSpec((D, D), lambda i: (0, 0))
    vec_spec = pl.BlockSpec((1, D), lambda i: (0, 0))
    return pl.pallas_call(
        functools.partial(_mlp_body, final_act=final_act),
        grid=grid,
        in_specs=[
            pl.BlockSpec(memory_space=pltpu.SMEM),
            row_spec, part0_spec, part1_spec,
            full_spec, vec_spec, vec_spec, vec_spec, full_spec, vec_spec,
        ],
        out_specs=row_spec,
        out_shape=jax.ShapeDtypeStruct((N, D), jnp.float32),
        interpret=interpret,
    )(eps, x, parts, parts, waT, ba, g, be, wbT, bb)


def kernel(x, edge_index, eps1, W1a, b1a, g1, be1, W1b, b1b,
           eps2, W2a, b2a, g2, be2, W2b, b2b):
    src = edge_index[0]
    dst = edge_index[1]
    npad = E_PAD - E
    srcp = jnp.concatenate([src, jnp.zeros((npad,), jnp.int32)])
    dstp = jnp.concatenate([dst, jnp.full((npad,), N, jnp.int32)])
    srcp = srcp.reshape(NW, CHUNKS, K)
    dstp = dstp.reshape(NW, CHUNKS, K)

    e1 = jnp.reshape(eps1, (1,))
    e2 = jnp.reshape(eps2, (1,))
    vecs = [v.reshape(1, D) for v in (b1a, g1, be1, b1b, b2a, g2, be2, b2b)]
    b1a_, g1_, be1_, b1b_, b2a_, g2_, be2_, b2b_ = vecs

    sc_agg = _get_sc_agg()
    parts1 = sc_agg(x, srcp, dstp)
    h1 = _tc_layer(e1, x, parts1, W1a.T, b1a_, g1_, be1_, W1b.T, b1b_,
                   final_act=True)
    parts2 = sc_agg(h1, srcp, dstp)
    h2 = _tc_layer(e2, h1, parts2, W2a.T, b2a_, g2_, be2_, W2b.T, b2b_,
                   final_act=False)
    return h2
